# Initial kernel scaffold; baseline (speedup 1.0000x reference)
import jax
import jax.numpy as jnp
from jax.experimental import pallas as pl


def kernel(vectors, centers, neighbor_species, W):
    raise NotImplementedError("write your pallas kernel here")


# speedup vs baseline: 48.8091x; 48.8091x over previous
import functools

import jax
import jax.numpy as jnp
import numpy as np
from jax.experimental import pallas as pl
from jax.experimental.pallas import tpu as pltpu

N_NODES = 10000
E_TOTAL = 160000
L_MAX = 3
N_MAX = 8
N_SPECIES = 4
N_PSEUDO = 4
CUTOFF = 4.0
F = 512

EB = 2048
ACC_ROWS = N_NODES + 8


def _sh_channels(xh, yh, zh):
    c1 = 0.4886025119029199
    one = jnp.ones_like(xh)
    chans = [
        0.28209479177387814 * one,
        c1 * yh, c1 * zh, c1 * xh,
        1.0925484305920792 * xh * yh,
        1.0925484305920792 * yh * zh,
        0.31539156525252005 * (3.0 * zh * zh - 1.0),
        1.0925484305920792 * xh * zh,
        0.5462742152960396 * (xh * xh - yh * yh),
        0.5900435899266435 * yh * (3.0 * xh * xh - yh * yh),
        2.890611442640554 * xh * yh * zh,
        0.4570457994644658 * yh * (5.0 * zh * zh - 1.0),
        0.3731763325901154 * zh * (5.0 * zh * zh - 3.0),
        0.4570457994644658 * xh * (5.0 * zh * zh - 1.0),
        1.445305721320277 * zh * (xh * xh - yh * yh),
        0.5900435899266435 * xh * (xh * xh - 3.0 * yh * yh),
    ]
    return chans


def _tc_kernel(keys_ref, x_ref, y_ref, z_ref, s_ref, w_ref, out_ref, feat_ref):
    i = pl.program_id(0)

    @pl.when(i == 0)
    def _init():
        out_ref[...] = jnp.zeros_like(out_ref)

    x = x_ref[...].reshape(EB, 1)
    y = y_ref[...].reshape(EB, 1)
    z = z_ref[...].reshape(EB, 1)
    sp = s_ref[...].reshape(EB, 1)

    r2 = x * x + y * y + z * z
    r = jnp.sqrt(r2)
    eps = 1e-12
    inv = 1.0 / (r + eps)
    xh, yh, zh = x * inv, y * inv, z * inv

    col = jax.lax.broadcasted_iota(jnp.int32, (1, F), 1)
    lm = col // (N_PSEUDO * N_MAX)
    p = (col // N_MAX) % N_PSEUDO
    n = col % N_MAX

    mu = n.astype(jnp.float32) * (CUTOFF / (N_MAX - 1))
    sigma = CUTOFF / N_MAX
    g = jnp.exp(-((r - mu) ** 2) / (2.0 * sigma * sigma))
    fc = jnp.where(r < CUTOFF, 0.5 * (jnp.cos(np.pi * r / CUTOFF) + 1.0), 0.0)
    radial = g * fc

    chans = _sh_channels(xh, yh, zh)
    sh = jnp.zeros((EB, F), jnp.float32)
    for k in range(16):
        sh = jnp.where(lm == k, chans[k], sh)

    pw = jnp.zeros((EB, F), jnp.float32)
    for s_val in range(N_SPECIES):
        wrow = jnp.zeros((1, F), jnp.float32)
        for p_val in range(N_PSEUDO):
            wrow = jnp.where(p == p_val, w_ref[p_val, s_val], wrow)
        pw = jnp.where(sp == s_val, wrow, pw)

    feat_ref[...] = sh * radial * pw

    def body(e, _):
        k = keys_ref[e]
        out_ref[pl.ds(k, 1), :] += feat_ref[pl.ds(e, 1), :]
        return 0

    jax.lax.fori_loop(0, EB, body, 0, unroll=4)


@jax.jit
def kernel(vectors, centers, neighbor_species, W):
    E = vectors.shape[0]
    Ep = ((E + EB - 1) // EB) * EB
    pad = Ep - E
    nb = Ep // EB
    xs = jnp.pad(vectors[:, 0], (0, pad))
    ys = jnp.pad(vectors[:, 1], (0, pad))
    zs = jnp.pad(vectors[:, 2], (0, pad))
    keys = jnp.pad(centers.astype(jnp.int32), (0, pad),
                   constant_values=N_NODES)
    sp = jnp.pad(neighbor_species.astype(jnp.int32), (0, pad))

    grid = (nb,)
    out = pl.pallas_call(
        _tc_kernel,
        grid=grid,
        in_specs=[
            pl.BlockSpec((EB,), lambda i: (i,), memory_space=pltpu.SMEM),
            pl.BlockSpec((EB,), lambda i: (i,)),
            pl.BlockSpec((EB,), lambda i: (i,)),
            pl.BlockSpec((EB,), lambda i: (i,)),
            pl.BlockSpec((EB,), lambda i: (i,)),
            pl.BlockSpec((N_PSEUDO, N_SPECIES), lambda i: (0, 0),
                         memory_space=pltpu.SMEM),
        ],
        out_specs=pl.BlockSpec((ACC_ROWS, F), lambda i: (0, 0)),
        out_shape=jax.ShapeDtypeStruct((ACC_ROWS, F), jnp.float32),
        scratch_shapes=[pltpu.VMEM((EB, F), jnp.float32)],
        compiler_params=pltpu.CompilerParams(
            dimension_semantics=("arbitrary",),
        ),
    )(keys, xs, ys, zs, sp, W)
    return out[:N_NODES]


# speedup vs baseline: 97.4591x; 1.9967x over previous
import functools

import jax
import jax.numpy as jnp
import numpy as np
from jax import lax
from jax.experimental import pallas as pl
from jax.experimental.pallas import tpu as pltpu
from jax.experimental.pallas import tpu_sc as plsc

N_NODES = 10000
L_MAX = 3
N_MAX = 8
N_SPECIES = 4
N_PSEUDO = 4
CUTOFF = 4.0
F = 512
FK = 128

EB = 2048
NS_TILES = 16
G = 64
TRASH = N_NODES
ACC_R = 10112
ZR = 640
CROWS = 400


def _sh_channels(xh, yh, zh):
    c1 = 0.4886025119029199
    one = jnp.ones_like(xh)
    return [
        0.28209479177387814 * one,
        c1 * yh, c1 * zh, c1 * xh,
        1.0925484305920792 * xh * yh,
        1.0925484305920792 * yh * zh,
        0.31539156525252005 * (3.0 * zh * zh - 1.0),
        1.0925484305920792 * xh * zh,
        0.5462742152960396 * (xh * xh - yh * yh),
        0.5900435899266435 * yh * (3.0 * xh * xh - yh * yh),
        2.890611442640554 * xh * yh * zh,
        0.4570457994644658 * yh * (5.0 * zh * zh - 1.0),
        0.3731763325901154 * zh * (5.0 * zh * zh - 3.0),
        0.4570457994644658 * xh * (5.0 * zh * zh - 1.0),
        1.445305721320277 * zh * (xh * xh - yh * yh),
        0.5900435899266435 * xh * (xh * xh - 3.0 * yh * yh),
    ]


def _feat_kernel(x_ref, y_ref, z_ref, feat_ref):
    x = x_ref[...].reshape(EB, 1)
    y = y_ref[...].reshape(EB, 1)
    z = z_ref[...].reshape(EB, 1)

    r2 = x * x + y * y + z * z
    r = jnp.sqrt(r2)
    inv = 1.0 / (r + 1e-12)
    xh, yh, zh = x * inv, y * inv, z * inv

    col = lax.broadcasted_iota(jnp.int32, (1, FK), 1)
    lm = col // N_MAX
    n = col % N_MAX

    mu = n.astype(jnp.float32) * (CUTOFF / (N_MAX - 1))
    sigma = CUTOFF / N_MAX
    g = jnp.exp(-((r - mu) ** 2) / (2.0 * sigma * sigma))
    fc = jnp.where(r < CUTOFF, 0.5 * (jnp.cos(np.pi * r / CUTOFF) + 1.0), 0.0)

    chans = _sh_channels(xh, yh, zh)
    sh = jnp.zeros((EB, FK), jnp.float32)
    for k in range(16):
        sh = jnp.where(lm == k, chans[k], sh)

    feat_ref[...] = sh * g * fc


def _stage_a(xs, ys, zs):
    Ep = xs.shape[0]
    nb = Ep // EB
    return pl.pallas_call(
        _feat_kernel,
        grid=(nb,),
        in_specs=[pl.BlockSpec((EB,), lambda i: (i,))] * 3,
        out_specs=pl.BlockSpec((EB, FK), lambda i: (i, 0)),
        out_shape=jax.ShapeDtypeStruct((Ep, FK), jnp.float32),
    )(xs, ys, zs)


def _sc_body(feat_hbm, cen_hbm, sp_hbm, zeros_hbm, a_hbm,
             cen_v, sp_v, l0_v, l1_v, idxg_v, ckey_v, feat_v, acc_sh, sem):
    E = cen_hbm.shape[0]
    ept = E // NS_TILES
    c = lax.axis_index("c")
    w = lax.axis_index("s")
    ebase = w * ept
    lane = lax.iota(jnp.int32, 16)

    pltpu.sync_copy(cen_hbm.at[pl.ds(ebase, ept)], cen_v)
    pltpu.sync_copy(sp_hbm.at[pl.ds(ebase, ept)], sp_v)

    def cls_body(i, offs):
        o0, o1 = offs
        s16 = sp_v[pl.ds(i * 16, 16)]
        idx16 = lane + i * 16
        m0 = s16 == 2 * c
        m1 = s16 == 2 * c + 1
        cs0 = plsc.cumsum(m0.astype(jnp.int32))
        cs1 = plsc.cumsum(m1.astype(jnp.int32))
        plsc.store_scatter(l0_v, [o0 + cs0 - 1], idx16, mask=m0)
        plsc.store_scatter(l1_v, [o1 + cs1 - 1], idx16, mask=m1)
        return (o0 + jnp.sum(m0.astype(jnp.int32)),
                o1 + jnp.sum(m1.astype(jnp.int32)))

    cnt0, cnt1 = lax.fori_loop(0, ept // 16, cls_body,
                               (jnp.int32(0), jnp.int32(0)))

    zrows = ACC_R // NS_TILES
    orows = 624

    def process(l_v, cnt):
        ngroups = (cnt + G - 1) // G

        def grp(g_i, _):
            gb = g_i * G
            for k in range(G // 16):
                pos = gb + k * 16 + lane
                valid = pos < cnt
                li = l_v[pl.ds(gb + k * 16, 16)]
                li = jnp.where(valid, li, 0)
                cv = plsc.load_gather(cen_v, [li])
                ckey_v[pl.ds(k * 16, 16)] = jnp.where(valid, cv, TRASH)
                idxg_v[pl.ds(k * 16, 16)] = jnp.where(valid, li + ebase, 0)
            pltpu.async_copy(feat_hbm.at[idxg_v], feat_v, sem).wait()
            pltpu.sync_copy(feat_v, acc_sh.at[ckey_v], add=True)
            return 0

        lax.fori_loop(0, ngroups, grp, 0)

    for j in range(2):
        plsc.subcore_barrier()
        pltpu.sync_copy(zeros_hbm.at[pl.ds(0, zrows)],
                        acc_sh.at[pl.ds(w * zrows, zrows)])
        plsc.subcore_barrier()
        process(l0_v if j == 0 else l1_v, cnt0 if j == 0 else cnt1)
        plsc.subcore_barrier()
        s_id = 2 * c + j
        pltpu.sync_copy(acc_sh.at[pl.ds(w * orows, orows)],
                        a_hbm.at[pl.ds(s_id * N_NODES + w * orows, orows)])
        rem = N_NODES - NS_TILES * orows
        @pl.when(w == 0)
        def _tail():
            pltpu.sync_copy(
                acc_sh.at[pl.ds(NS_TILES * orows, rem)],
                a_hbm.at[pl.ds(s_id * N_NODES + NS_TILES * orows, rem)])


def _stage_b(feat, cen, sp, zeros):
    mesh = plsc.VectorSubcoreMesh(core_axis_name="c", subcore_axis_name="s")
    lcap = ((cen.shape[0] // NS_TILES + G - 1) // G) * G
    return pl.kernel(
        _sc_body,
        out_type=jax.ShapeDtypeStruct((N_SPECIES * N_NODES, FK), jnp.float32),
        mesh=mesh,
        scratch_types=[
            pltpu.VMEM((cen.shape[0] // NS_TILES,), jnp.int32),
            pltpu.VMEM((cen.shape[0] // NS_TILES,), jnp.int32),
            pltpu.VMEM((lcap,), jnp.int32),
            pltpu.VMEM((lcap,), jnp.int32),
            pltpu.VMEM((G,), jnp.int32),
            pltpu.VMEM((G,), jnp.int32),
            pltpu.VMEM((G, FK), jnp.float32),
            pltpu.VMEM_SHARED((ACC_R, FK), jnp.float32),
            pltpu.SemaphoreType.DMA,
        ],
        compiler_params=pltpu.CompilerParams(needs_layout_passes=False),
    )(feat, cen, sp, zeros)


def _mix_kernel(a_ref, m_ref, out_ref):
    a2 = jnp.concatenate([a_ref[s] for s in range(N_SPECIES)], axis=1)
    out_ref[...] = jnp.dot(a2, m_ref[...], preferred_element_type=jnp.float32)


def _stage_c(a, m):
    nb = N_NODES // CROWS
    return pl.pallas_call(
        _mix_kernel,
        grid=(nb,),
        in_specs=[
            pl.BlockSpec((N_SPECIES, CROWS, FK), lambda i: (0, i, 0)),
            pl.BlockSpec((F, F), lambda i: (0, 0)),
        ],
        out_specs=pl.BlockSpec((CROWS, F), lambda i: (i, 0)),
        out_shape=jax.ShapeDtypeStruct((N_NODES, F), jnp.float32),
    )(a, m)


@jax.jit
def kernel(vectors, centers, neighbor_species, W):
    E = vectors.shape[0]
    Ep = ((E + EB - 1) // EB) * EB
    pad = Ep - E
    xs = jnp.pad(vectors[:, 0], (0, pad))
    ys = jnp.pad(vectors[:, 1], (0, pad))
    zs = jnp.pad(vectors[:, 2], (0, pad))
    cen = centers.astype(jnp.int32)
    sp = neighbor_species.astype(jnp.int32)

    feat = _stage_a(xs, ys, zs)

    zeros = jnp.zeros((ZR, FK), jnp.float32)
    a = _stage_b(feat, cen, sp, zeros)

    m = jnp.einsum("ps,lL,nN->slnLpN", W.astype(jnp.float32),
                   jnp.eye(16, dtype=jnp.float32),
                   jnp.eye(8, dtype=jnp.float32)).reshape(F, F)

    return _stage_c(a.reshape(N_SPECIES, N_NODES, FK), m)
